# baseline (speedup 1.0000x reference)
import math

import jax
import jax.numpy as jnp
from jax.experimental import pallas as pl

_B, _N, _C = 2, 4096, 1024
_SUB = 512
_SCALE = 1.0 / math.sqrt(_C)


def _attn_body(nb, x_ref, w_ref, os_ref, d_ref):
    xg = x_ref[...][:, :, :_C].reshape(nb * _SUB, _C)
    qkv = jnp.dot(xg, w_ref[...], preferred_element_type=jnp.float32)
    qkvb = qkv.astype(jnp.bfloat16)
    for b in range(nb):
        q = qkvb[b * _SUB:(b + 1) * _SUB, :_C]
        k = qkvb[b * _SUB:(b + 1) * _SUB, _C:2 * _C]
        v = qkvb[b * _SUB:(b + 1) * _SUB, 2 * _C:]
        s = jax.lax.dot_general(
            q, k, (((1,), (1,)), ((), ())), preferred_element_type=jnp.float32
        ) * _SCALE
        e = jnp.exp(s)
        d = jnp.sum(e, axis=-1, keepdims=True)
        p = (e * (1.0 / d)).astype(jnp.bfloat16)
        os = jax.lax.dot_general(
            p, v, (((1,), (0,)), ((), ())), preferred_element_type=jnp.float32
        )
        os_ref[b] = os
        d_ref[b] = d


def _run_part(xv, w, nseg, r, batch_merge, interpret=False):
    lanes = r * _C
    if batch_merge:
        grid = (nseg,)
        nb = _B
        x_spec = pl.BlockSpec((_B, _SUB, lanes), lambda s: (0, s, 0))
        w_spec = pl.BlockSpec((_C, 3 * _C), lambda s: (0, 0))
        os_spec = pl.BlockSpec((_B, _SUB, _C), lambda s: (0, s, 0))
        d_spec = pl.BlockSpec((_B, _SUB, 1), lambda s: (0, s, 0))
    else:
        grid = (_B, nseg)
        nb = 1
        x_spec = pl.BlockSpec((1, _SUB, lanes), lambda b, s: (b, s, 0))
        w_spec = pl.BlockSpec((_C, 3 * _C), lambda b, s: (0, 0))
        os_spec = pl.BlockSpec((1, _SUB, _C), lambda b, s: (b, s, 0))
        d_spec = pl.BlockSpec((1, _SUB, 1), lambda b, s: (b, s, 0))
    body = lambda *refs: _attn_body(nb, *refs)
    return pl.pallas_call(
        body,
        grid=grid,
        in_specs=[x_spec, w_spec],
        out_specs=[os_spec, d_spec],
        out_shape=[
            jax.ShapeDtypeStruct((_B, nseg * _SUB, _C), jnp.float32),
            jax.ShapeDtypeStruct((_B, nseg * _SUB, 1), jnp.float32),
        ],
        interpret=interpret,
    )(xv, w)


def _mix_body(osa_ref, da_ref, osb_ref, db_ref, osc_ref, dc_ref, out_ref):
    da = da_ref[0]
    db = db_ref[0]
    dc = dc_ref[0]
    osa = osa_ref[0]
    osb = osb_ref[0]
    osc = osc_ref[0]
    chunks = []
    for j in range(8):
        aj = da[:, j:j + 1]
        num = aj * osa[:, j * _C:(j + 1) * _C]
        dsj = aj
        if j % 2 == 0:
            h = j // 2
            bh = db[:, h:h + 1]
            num = num + bh * osb[:, h * _C:(h + 1) * _C]
            dsj = dsj + bh
        if j == 0:
            num = num + dc * osc
            dsj = dsj + dc
        chunks.append(num * (1.0 / dsj))
    out_ref[0] = jnp.concatenate(chunks, axis=1)


def _mix(osa, da, osb, db, osc, dc, interpret=False):
    rows = _N // 8
    blk = 64
    out = pl.pallas_call(
        _mix_body,
        grid=(_B, rows // blk),
        in_specs=[
            pl.BlockSpec((1, blk, 8 * _C), lambda b, k: (b, k, 0)),
            pl.BlockSpec((1, blk, 8), lambda b, k: (b, k, 0)),
            pl.BlockSpec((1, blk, 4 * _C), lambda b, k: (b, k, 0)),
            pl.BlockSpec((1, blk, 4), lambda b, k: (b, k, 0)),
            pl.BlockSpec((1, blk, _C), lambda b, k: (b, k, 0)),
            pl.BlockSpec((1, blk, 1), lambda b, k: (b, k, 0)),
        ],
        out_specs=pl.BlockSpec((1, blk, 8 * _C), lambda b, k: (b, k, 0)),
        out_shape=jax.ShapeDtypeStruct((_B, rows, 8 * _C), jnp.float32),
        interpret=interpret,
    )(
        osa.reshape(_B, rows, 8 * _C),
        da.reshape(_B, rows, 8),
        osb.reshape(_B, rows, 4 * _C),
        db.reshape(_B, rows, 4),
        osc,
        dc,
    )
    return out.reshape(_B, _N, _C)


def _dilated_attention(x, wq, wk, wv, interpret=False):
    xb = x.astype(jnp.bfloat16)
    w = jnp.concatenate([wq, wk, wv], axis=1).astype(jnp.bfloat16)
    osa, da = _run_part(xb, w, 8, 1, True, interpret)
    osb, db = _run_part(
        xb.reshape(_B, _N // 2, 2 * _C), w, 4, 2, True, interpret
    )
    osc, dc = _run_part(
        xb.reshape(_B, _N // 8, 8 * _C), w, 1, 8, False, interpret
    )
    return _mix(osa, da, osb, db, osc, dc, interpret)


def kernel(x, Wq, Wk, Wv):
    return _dilated_attention(x, Wq, Wk, Wv)


# speedup vs baseline: 1.0034x; 1.0034x over previous
import math

import jax
import jax.numpy as jnp
from jax.experimental import pallas as pl

_B, _N, _C = 2, 4096, 1024
_SUB = 512
_SCALE = 1.0 / math.sqrt(_C)


def _attn_body(nb, x_ref, w_ref, os_ref, d_ref):
    xg = x_ref[...][:, :, :_C].reshape(nb * _SUB, _C)
    qkv = jnp.dot(xg, w_ref[...], preferred_element_type=jnp.float32)
    qkvb = qkv.astype(jnp.bfloat16)
    for b in range(nb):
        q = qkvb[b * _SUB:(b + 1) * _SUB, :_C]
        k = qkvb[b * _SUB:(b + 1) * _SUB, _C:2 * _C]
        v = qkvb[b * _SUB:(b + 1) * _SUB, 2 * _C:]
        s = jax.lax.dot_general(
            q, k, (((1,), (1,)), ((), ())), preferred_element_type=jnp.float32
        ) * _SCALE
        e = jnp.exp(s)
        d = jnp.sum(e, axis=-1, keepdims=True)
        p = (e * (1.0 / d)).astype(jnp.bfloat16)
        os = jax.lax.dot_general(
            p, v, (((1,), (0,)), ((), ())), preferred_element_type=jnp.float32
        )
        os_ref[b] = os
        d_ref[b] = d


def _run_part(xv, w, nseg, r, batch_merge, interpret=False):
    lanes = r * _C
    if batch_merge:
        grid = (nseg,)
        nb = _B
        x_spec = pl.BlockSpec((_B, _SUB, lanes), lambda s: (0, s, 0))
        w_spec = pl.BlockSpec((_C, 3 * _C), lambda s: (0, 0))
        os_spec = pl.BlockSpec((_B, _SUB, _C), lambda s: (0, s, 0))
        d_spec = pl.BlockSpec((_B, _SUB, 1), lambda s: (0, s, 0))
    else:
        grid = (_B, nseg)
        nb = 1
        x_spec = pl.BlockSpec((1, _SUB, lanes), lambda b, s: (b, s, 0))
        w_spec = pl.BlockSpec((_C, 3 * _C), lambda b, s: (0, 0))
        os_spec = pl.BlockSpec((1, _SUB, _C), lambda b, s: (b, s, 0))
        d_spec = pl.BlockSpec((1, _SUB, 1), lambda b, s: (b, s, 0))
    body = lambda *refs: _attn_body(nb, *refs)
    return pl.pallas_call(
        body,
        grid=grid,
        in_specs=[x_spec, w_spec],
        out_specs=[os_spec, d_spec],
        out_shape=[
            jax.ShapeDtypeStruct((_B, nseg * _SUB, _C), jnp.float32),
            jax.ShapeDtypeStruct((_B, nseg * _SUB, 1), jnp.float32),
        ],
        interpret=interpret,
    )(xv, w)


def _mix_body(osa_ref, da_ref, osb_ref, db_ref, osc_ref, dc_ref, out_ref):
    da = da_ref[0]
    db = db_ref[0]
    dc = dc_ref[0]
    osa = osa_ref[0]
    osb = osb_ref[0]
    osc = osc_ref[0]
    chunks = []
    for j in range(8):
        aj = da[:, j:j + 1]
        num = aj * osa[:, j * _C:(j + 1) * _C]
        dsj = aj
        if j % 2 == 0:
            h = j // 2
            bh = db[:, h:h + 1]
            num = num + bh * osb[:, h * _C:(h + 1) * _C]
            dsj = dsj + bh
        if j == 0:
            num = num + dc * osc
            dsj = dsj + dc
        chunks.append(num * (1.0 / dsj))
    out_ref[0] = jnp.concatenate(chunks, axis=1)


def _mix(osa, da, osb, db, osc, dc, interpret=False):
    rows = _N // 8
    blk = 64
    out = pl.pallas_call(
        _mix_body,
        grid=(_B, rows // blk),
        in_specs=[
            pl.BlockSpec((1, blk, 8 * _C), lambda b, k: (b, k, 0)),
            pl.BlockSpec((1, blk, 8), lambda b, k: (b, k, 0)),
            pl.BlockSpec((1, blk, 4 * _C), lambda b, k: (b, k, 0)),
            pl.BlockSpec((1, blk, 4), lambda b, k: (b, k, 0)),
            pl.BlockSpec((1, blk, _C), lambda b, k: (b, k, 0)),
            pl.BlockSpec((1, blk, 1), lambda b, k: (b, k, 0)),
        ],
        out_specs=pl.BlockSpec((1, blk, 8 * _C), lambda b, k: (b, k, 0)),
        out_shape=jax.ShapeDtypeStruct((_B, rows, 8 * _C), jnp.float32),
        interpret=interpret,
    )(
        osa.reshape(_B, rows, 8 * _C),
        da.reshape(_B, rows, 8),
        osb.reshape(_B, rows, 4 * _C),
        db.reshape(_B, rows, 4),
        osc,
        dc,
    )
    return out.reshape(_B, _N, _C)


def _dilated_attention(x, wq, wk, wv, interpret=False):
    xb = x.astype(jnp.bfloat16)
    w = jnp.concatenate([wq, wk, wv], axis=1).astype(jnp.bfloat16)
    osa, da = _run_part(xb, w, 8, 1, False, interpret)
    osb, db = _run_part(
        xb.reshape(_B, _N // 2, 2 * _C), w, 4, 2, False, interpret
    )
    osc, dc = _run_part(
        xb.reshape(_B, _N // 8, 8 * _C), w, 1, 8, False, interpret
    )
    return _mix(osa, da, osb, db, osc, dc, interpret)


def kernel(x, Wq, Wk, Wv):
    return _dilated_attention(x, Wq, Wk, Wv)


# speedup vs baseline: 1.3558x; 1.3511x over previous
import math

import jax
import jax.numpy as jnp
from jax.experimental import pallas as pl

_B, _N, _C = 2, 4096, 1024
_SUB = 512
_SCALE = 1.0 / math.sqrt(_C)


def _attn_body(nb, x_ref, w_ref, os_ref, d_ref):
    xg = x_ref[...][:, :, :_C].reshape(nb * _SUB, _C)
    qkv = jnp.dot(xg, w_ref[...], preferred_element_type=jnp.float32)
    qkvb = qkv.astype(jnp.bfloat16)
    for b in range(nb):
        q = qkvb[b * _SUB:(b + 1) * _SUB, :_C]
        k = qkvb[b * _SUB:(b + 1) * _SUB, _C:2 * _C]
        v = qkvb[b * _SUB:(b + 1) * _SUB, 2 * _C:]
        s = jax.lax.dot_general(
            q, k, (((1,), (1,)), ((), ())), preferred_element_type=jnp.float32
        ) * _SCALE
        e = jnp.exp(s)
        d = jnp.sum(e, axis=-1, keepdims=True)
        p = (e * (1.0 / d)).astype(jnp.bfloat16)
        os = jax.lax.dot_general(
            p, v, (((1,), (0,)), ((), ())), preferred_element_type=jnp.float32
        )
        os_ref[b] = os.astype(jnp.bfloat16)
        d_ref[b] = d


def _run_part(xv, w, nseg, r, batch_merge, interpret=False):
    lanes = r * _C
    if batch_merge:
        grid = (nseg,)
        nb = _B
        x_spec = pl.BlockSpec((_B, _SUB, lanes), lambda s: (0, s, 0))
        w_spec = pl.BlockSpec((_C, 3 * _C), lambda s: (0, 0))
        os_spec = pl.BlockSpec((_B, _SUB, _C), lambda s: (0, s, 0))
        d_spec = pl.BlockSpec((_B, _SUB, 1), lambda s: (0, s, 0))
    else:
        grid = (_B, nseg)
        nb = 1
        x_spec = pl.BlockSpec((1, _SUB, lanes), lambda b, s: (b, s, 0))
        w_spec = pl.BlockSpec((_C, 3 * _C), lambda b, s: (0, 0))
        os_spec = pl.BlockSpec((1, _SUB, _C), lambda b, s: (b, s, 0))
        d_spec = pl.BlockSpec((1, _SUB, 1), lambda b, s: (b, s, 0))
    body = lambda *refs: _attn_body(nb, *refs)
    return pl.pallas_call(
        body,
        grid=grid,
        in_specs=[x_spec, w_spec],
        out_specs=[os_spec, d_spec],
        out_shape=[
            jax.ShapeDtypeStruct((_B, nseg * _SUB, _C), jnp.bfloat16),
            jax.ShapeDtypeStruct((_B, nseg * _SUB, 1), jnp.float32),
        ],
        interpret=interpret,
    )(xv, w)


def _mix_body(osa_ref, da_ref, osb_ref, db_ref, osc_ref, dc_ref, out_ref):
    da = da_ref[0]
    db = db_ref[0]
    dc = dc_ref[0]
    osa = osa_ref[0].astype(jnp.float32)
    osb = osb_ref[0].astype(jnp.float32)
    osc = osc_ref[0].astype(jnp.float32)
    i = jax.lax.broadcasted_iota(jnp.int32, (_SUB, 1), 0)
    m2 = (i % 2) == 0
    m8 = (i % 8) == 0
    nb = jnp.repeat(db * osb, 2, axis=0)
    dbr = jnp.repeat(db, 2, axis=0)
    nc = jnp.repeat(dc * osc, 8, axis=0)
    dcr = jnp.repeat(dc, 8, axis=0)
    num = da * osa + jnp.where(m2, nb, 0.0) + jnp.where(m8, nc, 0.0)
    ds = da + jnp.where(m2, dbr, 0.0) + jnp.where(m8, dcr, 0.0)
    out_ref[0] = num * (1.0 / ds)


def _mix(osa, da, osb, db, osc, dc, interpret=False):
    return pl.pallas_call(
        _mix_body,
        grid=(_B, _N // _SUB),
        in_specs=[
            pl.BlockSpec((1, _SUB, _C), lambda b, k: (b, k, 0)),
            pl.BlockSpec((1, _SUB, 1), lambda b, k: (b, k, 0)),
            pl.BlockSpec((1, _SUB // 2, _C), lambda b, k: (b, k, 0)),
            pl.BlockSpec((1, _SUB // 2, 1), lambda b, k: (b, k, 0)),
            pl.BlockSpec((1, _SUB // 8, _C), lambda b, k: (b, k, 0)),
            pl.BlockSpec((1, _SUB // 8, 1), lambda b, k: (b, k, 0)),
        ],
        out_specs=pl.BlockSpec((1, _SUB, _C), lambda b, k: (b, k, 0)),
        out_shape=jax.ShapeDtypeStruct((_B, _N, _C), jnp.float32),
        interpret=interpret,
    )(osa, da, osb, db, osc, dc)


def _dilated_attention(x, wq, wk, wv, interpret=False):
    xb = x.astype(jnp.bfloat16)
    w = jnp.concatenate([wq, wk, wv], axis=1).astype(jnp.bfloat16)
    osa, da = _run_part(xb, w, 8, 1, False, interpret)
    osb, db = _run_part(
        xb.reshape(_B, _N // 2, 2 * _C), w, 4, 2, False, interpret
    )
    osc, dc = _run_part(
        xb.reshape(_B, _N // 8, 8 * _C), w, 1, 8, False, interpret
    )
    return _mix(osa, da, osb, db, osc, dc, interpret)


def kernel(x, Wq, Wk, Wv):
    return _dilated_attention(x, Wq, Wk, Wv)


# speedup vs baseline: 1.7576x; 1.2964x over previous
import math

import jax
import jax.numpy as jnp
from jax.experimental import pallas as pl

_B, _N, _C = 2, 4096, 1024
_SUB = 512
_SCALE = 1.0 / math.sqrt(_C)


def _attention(qkvb):
    q = qkvb[:, :_C]
    k = qkvb[:, _C:2 * _C]
    v = qkvb[:, 2 * _C:]
    s = jax.lax.dot_general(
        q, k, (((1,), (1,)), ((), ())), preferred_element_type=jnp.float32
    ) * _SCALE
    e = jnp.exp(s)
    d = jnp.sum(e, axis=-1, keepdims=True)
    p = (e * (1.0 / d)).astype(jnp.bfloat16)
    os = jax.lax.dot_general(
        p, v, (((1,), (0,)), ((), ())), preferred_element_type=jnp.float32
    )
    return os.astype(jnp.bfloat16), d


def _attn_a_body(x_ref, w_ref, xb_ref, os_ref, d_ref):
    xg = x_ref[0].astype(jnp.bfloat16)
    xb_ref[0] = xg
    qkv = jnp.dot(xg, w_ref[...], preferred_element_type=jnp.float32)
    os, d = _attention(qkv.astype(jnp.bfloat16))
    os_ref[0] = os
    d_ref[0] = d


def _attn_a(x, w, interpret=False):
    return pl.pallas_call(
        _attn_a_body,
        grid=(_B, 8),
        in_specs=[
            pl.BlockSpec((1, _SUB, _C), lambda b, s: (b, s, 0)),
            pl.BlockSpec((_C, 3 * _C), lambda b, s: (0, 0)),
        ],
        out_specs=[
            pl.BlockSpec((1, _SUB, _C), lambda b, s: (b, s, 0)),
            pl.BlockSpec((1, _SUB, _C), lambda b, s: (b, s, 0)),
            pl.BlockSpec((1, _SUB, 1), lambda b, s: (b, s, 0)),
        ],
        out_shape=[
            jax.ShapeDtypeStruct((_B, _N, _C), jnp.bfloat16),
            jax.ShapeDtypeStruct((_B, _N, _C), jnp.bfloat16),
            jax.ShapeDtypeStruct((_B, _N, 1), jnp.float32),
        ],
        interpret=interpret,
    )(x, w)


def _gather_stride(blk, r):
    rows = jax.lax.broadcasted_iota(jnp.int32, (_SUB, r * _SUB), 0)
    cols = jax.lax.broadcasted_iota(jnp.int32, (_SUB, r * _SUB), 1)
    sel = (cols == r * rows).astype(jnp.bfloat16)
    g = jax.lax.dot_general(
        sel, blk, (((1,), (0,)), ((), ())), preferred_element_type=jnp.float32
    )
    return g.astype(jnp.bfloat16)


def _attn_bc_body(r, xb_ref, w_ref, os_ref, d_ref):
    xg = _gather_stride(xb_ref[0], r)
    qkv = jnp.dot(xg, w_ref[...], preferred_element_type=jnp.float32)
    os, d = _attention(qkv.astype(jnp.bfloat16))
    os_ref[0] = os
    d_ref[0] = d


def _attn_bc(xb, w, nseg, r, interpret=False):
    body = lambda *refs: _attn_bc_body(r, *refs)
    return pl.pallas_call(
        body,
        grid=(_B, nseg),
        in_specs=[
            pl.BlockSpec((1, r * _SUB, _C), lambda b, s: (b, s, 0)),
            pl.BlockSpec((_C, 3 * _C), lambda b, s: (0, 0)),
        ],
        out_specs=[
            pl.BlockSpec((1, _SUB, _C), lambda b, s: (b, s, 0)),
            pl.BlockSpec((1, _SUB, 1), lambda b, s: (b, s, 0)),
        ],
        out_shape=[
            jax.ShapeDtypeStruct((_B, nseg * _SUB, _C), jnp.bfloat16),
            jax.ShapeDtypeStruct((_B, nseg * _SUB, 1), jnp.float32),
        ],
        interpret=interpret,
    )(xb, w)


def _mix_body(osa_ref, da_ref, osb_ref, db_ref, osc_ref, dc_ref, out_ref):
    da = da_ref[0]
    db = db_ref[0]
    dc = dc_ref[0]
    osa = osa_ref[0].astype(jnp.float32)
    osb = osb_ref[0].astype(jnp.float32)
    osc = osc_ref[0].astype(jnp.float32)
    i = jax.lax.broadcasted_iota(jnp.int32, (_SUB, 1), 0)
    m2 = (i % 2) == 0
    m8 = (i % 8) == 0
    nb = jnp.repeat(db * osb, 2, axis=0)
    dbr = jnp.repeat(db, 2, axis=0)
    nc = jnp.repeat(dc * osc, 8, axis=0)
    dcr = jnp.repeat(dc, 8, axis=0)
    num = da * osa + jnp.where(m2, nb, 0.0) + jnp.where(m8, nc, 0.0)
    ds = da + jnp.where(m2, dbr, 0.0) + jnp.where(m8, dcr, 0.0)
    out_ref[0] = num * (1.0 / ds)


def _mix(osa, da, osb, db, osc, dc, interpret=False):
    return pl.pallas_call(
        _mix_body,
        grid=(_B, _N // _SUB),
        in_specs=[
            pl.BlockSpec((1, _SUB, _C), lambda b, k: (b, k, 0)),
            pl.BlockSpec((1, _SUB, 1), lambda b, k: (b, k, 0)),
            pl.BlockSpec((1, _SUB // 2, _C), lambda b, k: (b, k, 0)),
            pl.BlockSpec((1, _SUB // 2, 1), lambda b, k: (b, k, 0)),
            pl.BlockSpec((1, _SUB // 8, _C), lambda b, k: (b, k, 0)),
            pl.BlockSpec((1, _SUB // 8, 1), lambda b, k: (b, k, 0)),
        ],
        out_specs=pl.BlockSpec((1, _SUB, _C), lambda b, k: (b, k, 0)),
        out_shape=jax.ShapeDtypeStruct((_B, _N, _C), jnp.float32),
        interpret=interpret,
    )(osa, da, osb, db, osc, dc)


def _dilated_attention(x, wq, wk, wv, interpret=False):
    w = jnp.concatenate([wq, wk, wv], axis=1).astype(jnp.bfloat16)
    xb, osa, da = _attn_a(x, w, interpret)
    osb, db = _attn_bc(xb, w, 4, 2, interpret)
    osc, dc = _attn_bc(xb, w, 1, 8, interpret)
    return _mix(osa, da, osb, db, osc, dc, interpret)


def kernel(x, Wq, Wk, Wv):
    return _dilated_attention(x, Wq, Wk, Wv)
